# Initial kernel scaffold; baseline (speedup 1.0000x reference)
import jax
import jax.numpy as jnp
from jax.experimental import pallas as pl


def kernel(in_feature, nbr_idx, weight, bias):
    raise NotImplementedError("write your pallas kernel here")


# speedup vs baseline: 1.9781x; 1.9781x over previous
import functools

import jax
import jax.numpy as jnp
from jax import lax
from jax.experimental import pallas as pl
from jax.experimental.pallas import tpu as pltpu
from jax.experimental.pallas import tpu_sc as plsc

N = 50000
CIN = 32
COUT = 32
KVOL = 27

NW = 32
ROWS_PER_W = 1664
NPAD = NW * ROWS_PER_W
BLK = 512
NB = NPAD // BLK


def _tc_body(x_ref, nbr_ref, w_ref, b_ref, y_ref, idx_ref):
    x = x_ref[...]
    for k in range(KVOL):
        y = jnp.dot(x, w_ref[k], preferred_element_type=jnp.float32)
        if k == 0:
            y = y + b_ref[...]
        y_ref[k] = y
    offs = lax.broadcasted_iota(jnp.int32, (KVOL, BLK), 0) * NPAD
    idx_ref[...] = nbr_ref[...] + offs


def _tc_gemm(x_pad, nbr_pad, w, bias2d):
    return pl.pallas_call(
        _tc_body,
        grid=(NB,),
        in_specs=[
            pl.BlockSpec((BLK, CIN), lambda i: (i, 0)),
            pl.BlockSpec((KVOL, BLK), lambda i: (0, i)),
            pl.BlockSpec((KVOL, CIN, COUT), lambda i: (0, 0, 0)),
            pl.BlockSpec((1, COUT), lambda i: (0, 0)),
        ],
        out_specs=[
            pl.BlockSpec((KVOL, BLK, COUT), lambda i: (0, i, 0)),
            pl.BlockSpec((KVOL, BLK), lambda i: (0, i)),
        ],
        out_shape=[
            jax.ShapeDtypeStruct((KVOL, NPAD, COUT), jnp.float32),
            jax.ShapeDtypeStruct((KVOL, NPAD), jnp.int32),
        ],
    )(x_pad, nbr_pad, w, bias2d)


def _sc_gather_sum(y, idx):
    nc = 2
    mesh = plsc.VectorSubcoreMesh(
        core_axis_name="c", subcore_axis_name="s", num_cores=nc, num_subcores=16
    )

    y2d = y.reshape(KVOL * NPAD, COUT)
    idx1d = idx.reshape(KVOL * NPAD)

    @functools.partial(
        pl.kernel,
        out_type=jax.ShapeDtypeStruct((NPAD, COUT), jnp.float32),
        mesh=mesh,
        scratch_types=[
            pltpu.VMEM((ROWS_PER_W,), jnp.int32),
            pltpu.VMEM((ROWS_PER_W, COUT), jnp.float32),
            pltpu.SemaphoreType.DMA,
        ],
        compiler_params=pltpu.CompilerParams(use_tc_tiling_on_sc=False),
    )
    def sc_k(y_hbm, idx_hbm, out_hbm, idx_v, acc_v, sem):
        wid = lax.axis_index("s") * nc + lax.axis_index("c")
        base = wid * ROWS_PER_W
        for k in range(KVOL):
            pltpu.sync_copy(idx_hbm.at[pl.ds(k * NPAD + base, ROWS_PER_W)], idx_v)
            pltpu.async_copy(
                y_hbm.at[idx_v], acc_v, sem, add=(k > 0)
            ).wait()
        pltpu.sync_copy(acc_v, out_hbm.at[pl.ds(base, ROWS_PER_W)])

    return sc_k(y2d, idx1d)


def kernel(in_feature, nbr_idx, weight, bias):
    w = weight.reshape(COUT, CIN, KVOL).transpose(2, 1, 0)
    x_pad = jnp.pad(in_feature, ((0, NPAD - N), (0, 0)))
    nbr_pad = jnp.pad(nbr_idx, ((0, 0), (0, NPAD - N)))
    y, idx = _tc_gemm(x_pad, nbr_pad, w, bias.reshape(1, COUT))
    out = _sc_gather_sum(y, idx)
    return out[:N]


# speedup vs baseline: 2.2252x; 1.1249x over previous
import functools

import jax
import jax.numpy as jnp
from jax import lax
from jax.experimental import pallas as pl
from jax.experimental.pallas import tpu as pltpu
from jax.experimental.pallas import tpu_sc as plsc

N = 50000
CIN = 32
COUT = 32
KVOL = 27

NW = 32
ROWS_PER_W = 1664
NPAD = NW * ROWS_PER_W
BLK = 512
NB = NPAD // BLK


def _tc_body(x_ref, nbr_ref, w_ref, b_ref, y_ref, idx_ref):
    x = x_ref[...]
    for k in range(KVOL):
        y = jnp.dot(x, w_ref[k], preferred_element_type=jnp.float32)
        if k == 0:
            y = y + b_ref[...]
        y_ref[k] = y
    offs = lax.broadcasted_iota(jnp.int32, (KVOL, BLK), 0) * NPAD
    idx_ref[...] = nbr_ref[...] + offs


def _tc_gemm(x_pad, nbr_pad, w, bias2d):
    return pl.pallas_call(
        _tc_body,
        grid=(NB,),
        in_specs=[
            pl.BlockSpec((BLK, CIN), lambda i: (i, 0)),
            pl.BlockSpec((KVOL, BLK), lambda i: (0, i)),
            pl.BlockSpec((KVOL, CIN, COUT), lambda i: (0, 0, 0)),
            pl.BlockSpec((1, COUT), lambda i: (0, 0)),
        ],
        out_specs=[
            pl.BlockSpec((KVOL, BLK, COUT), lambda i: (0, i, 0)),
            pl.BlockSpec((KVOL, BLK), lambda i: (0, i)),
        ],
        out_shape=[
            jax.ShapeDtypeStruct((KVOL, NPAD, COUT), jnp.float32),
            jax.ShapeDtypeStruct((KVOL, NPAD), jnp.int32),
        ],
    )(x_pad, nbr_pad, w, bias2d)


def _sc_gather_sum(y, idx):
    nc = 2
    mesh = plsc.VectorSubcoreMesh(
        core_axis_name="c", subcore_axis_name="s", num_cores=nc, num_subcores=16
    )

    y2d = y.reshape(KVOL * NPAD, COUT)
    idx1d = idx.reshape(KVOL * NPAD)

    @functools.partial(
        pl.kernel,
        out_type=jax.ShapeDtypeStruct((NPAD, COUT), jnp.float32),
        mesh=mesh,
        scratch_types=[
            pltpu.VMEM((KVOL * ROWS_PER_W,), jnp.int32),
            pltpu.VMEM((ROWS_PER_W, COUT), jnp.float32),
            pltpu.SemaphoreType.DMA,
            pltpu.SemaphoreType.DMA,
        ],
        compiler_params=pltpu.CompilerParams(use_tc_tiling_on_sc=False),
    )
    def sc_k(y_hbm, idx_hbm, out_hbm, idx_v, acc_v, sem, isem):
        wid = lax.axis_index("s") * nc + lax.axis_index("c")
        base = wid * ROWS_PER_W
        idescs = [
            pltpu.async_copy(
                idx_hbm.at[pl.ds(k * NPAD + base, ROWS_PER_W)],
                idx_v.at[pl.ds(k * ROWS_PER_W, ROWS_PER_W)],
                isem,
            )
            for k in range(KVOL)
        ]
        z = jnp.zeros((16,), jnp.float32)

        def zero_body(r, carry):
            acc_v[r, pl.ds(0, 16)] = z
            acc_v[r, pl.ds(16, 16)] = z
            return carry

        lax.fori_loop(0, ROWS_PER_W, zero_body, 0)
        for d in idescs:
            d.wait()
        descs = [
            pltpu.async_copy(
                y_hbm.at[idx_v.at[pl.ds(k * ROWS_PER_W, ROWS_PER_W)]],
                acc_v,
                sem,
                add=True,
            )
            for k in range(KVOL)
        ]
        for d in descs:
            d.wait()
        pltpu.sync_copy(acc_v, out_hbm.at[pl.ds(base, ROWS_PER_W)])

    return sc_k(y2d, idx1d)


def kernel(in_feature, nbr_idx, weight, bias):
    w = weight.reshape(COUT, CIN, KVOL).transpose(2, 1, 0)
    x_pad = jnp.pad(in_feature, ((0, NPAD - N), (0, 0)))
    nbr_pad = jnp.pad(nbr_idx, ((0, 0), (0, NPAD - N)))
    y, idx = _tc_gemm(x_pad, nbr_pad, w, bias.reshape(1, COUT))
    out = _sc_gather_sum(y, idx)
    return out[:N]


# speedup vs baseline: 2.9421x; 1.3222x over previous
import functools

import jax
import jax.numpy as jnp
from jax import lax
from jax.experimental import pallas as pl
from jax.experimental.pallas import tpu as pltpu
from jax.experimental.pallas import tpu_sc as plsc

N = 50000
CIN = 32
COUT = 32
KVOL = 27
KGRP = 7
YW = 128

NW = 32
ROWS_PER_W = 1664
NPAD = NW * ROWS_PER_W
BLK = 512
NB = NPAD // BLK


def _tc_body(x_ref, nbr_ref, w_ref, b_ref, y_ref, idx_ref):
    x = x_ref[...]
    for q in range(KGRP):
        y = jnp.dot(x, w_ref[q], preferred_element_type=jnp.float32)
        if q == 0:
            y = y + b_ref[...]
        y_ref[q] = y
    ks = lax.broadcasted_iota(jnp.int32, (KVOL, BLK), 0)
    offs = (ks // 4) * (4 * NPAD) + ks % 4
    idx_ref[...] = nbr_ref[...] * 4 + offs


def _tc_gemm(x_pad, nbr_pad, w_stack, b128):
    return pl.pallas_call(
        _tc_body,
        grid=(NB,),
        in_specs=[
            pl.BlockSpec((BLK, CIN), lambda i: (i, 0)),
            pl.BlockSpec((KVOL, BLK), lambda i: (0, i)),
            pl.BlockSpec((KGRP, CIN, YW), lambda i: (0, 0, 0)),
            pl.BlockSpec((1, YW), lambda i: (0, 0)),
        ],
        out_specs=[
            pl.BlockSpec((KGRP, BLK, YW), lambda i: (0, i, 0)),
            pl.BlockSpec((KVOL, BLK), lambda i: (0, i)),
        ],
        out_shape=[
            jax.ShapeDtypeStruct((KGRP, NPAD, YW), jnp.float32),
            jax.ShapeDtypeStruct((KVOL, NPAD), jnp.int32),
        ],
    )(x_pad, nbr_pad, w_stack, b128)


def _sc_gather_sum(y, idx):
    nc = 2
    mesh = plsc.VectorSubcoreMesh(
        core_axis_name="c", subcore_axis_name="s", num_cores=nc, num_subcores=16
    )

    y2d = y.reshape(KGRP * NPAD * 4, COUT)
    idx1d = idx.reshape(KVOL * NPAD)

    @functools.partial(
        pl.kernel,
        out_type=jax.ShapeDtypeStruct((NPAD, COUT), jnp.float32),
        mesh=mesh,
        scratch_types=[
            pltpu.VMEM((KVOL * ROWS_PER_W,), jnp.int32),
            pltpu.VMEM((ROWS_PER_W, COUT), jnp.float32),
            pltpu.SemaphoreType.DMA,
            pltpu.SemaphoreType.DMA,
        ],
        compiler_params=pltpu.CompilerParams(use_tc_tiling_on_sc=False),
    )
    def sc_k(y_hbm, idx_hbm, out_hbm, idx_v, acc_v, sem, isem):
        wid = lax.axis_index("s") * nc + lax.axis_index("c")
        base = wid * ROWS_PER_W
        idescs = [
            pltpu.async_copy(
                idx_hbm.at[pl.ds(k * NPAD + base, ROWS_PER_W)],
                idx_v.at[pl.ds(k * ROWS_PER_W, ROWS_PER_W)],
                isem,
            )
            for k in range(KVOL)
        ]
        z = jnp.zeros((16,), jnp.float32)

        def zero_body(r, carry):
            acc_v[r, pl.ds(0, 16)] = z
            acc_v[r, pl.ds(16, 16)] = z
            return carry

        lax.fori_loop(0, ROWS_PER_W, zero_body, 0)
        for d in idescs:
            d.wait()
        descs = [
            pltpu.async_copy(
                y_hbm.at[idx_v.at[pl.ds(k * ROWS_PER_W, ROWS_PER_W)]],
                acc_v,
                sem,
                add=True,
            )
            for k in range(KVOL)
        ]
        for d in descs:
            d.wait()
        pltpu.sync_copy(acc_v, out_hbm.at[pl.ds(base, ROWS_PER_W)])

    return sc_k(y2d, idx1d)


def kernel(in_feature, nbr_idx, weight, bias):
    w = weight.reshape(COUT, CIN, KVOL).transpose(2, 1, 0)
    w_pad = jnp.pad(w, ((0, KGRP * 4 - KVOL), (0, 0), (0, 0)))
    w_stack = (
        w_pad.reshape(KGRP, 4, CIN, COUT)
        .transpose(0, 2, 1, 3)
        .reshape(KGRP, CIN, YW)
    )
    b128 = jnp.pad(bias.reshape(1, COUT), ((0, 0), (0, YW - COUT)))
    x_pad = jnp.pad(in_feature, ((0, NPAD - N), (0, 0)))
    nbr_pad = jnp.pad(nbr_idx, ((0, 0), (0, NPAD - N)))
    y, idx = _tc_gemm(x_pad, nbr_pad, w_stack, b128)
    out = _sc_gather_sum(y, idx)
    return out[:N]


# speedup vs baseline: 3.9970x; 1.3585x over previous
import functools

import jax
import jax.numpy as jnp
from jax import lax
from jax.experimental import pallas as pl
from jax.experimental.pallas import tpu as pltpu
from jax.experimental.pallas import tpu_sc as plsc

N = 50000
CIN = 32
COUT = 32
KVOL = 27
PACK = 4
YW = PACK * COUT

NW = 32
ROWS_PER_W = 1664
NPAD = NW * ROWS_PER_W
BLK = 512
NB = NPAD // PACK // BLK


def _tc_body(x_ref, nbr_ref, w_ref, b_ref, y_ref, idx_ref):
    x = x_ref[...]
    for k in range(KVOL):
        y = jnp.dot(x, w_ref[k], preferred_element_type=jnp.float32)
        if k == 0:
            y = y + b_ref[...]
        y_ref[k] = y
    offs = lax.broadcasted_iota(jnp.int32, (KVOL, PACK * BLK), 0) * NPAD
    idx_ref[...] = nbr_ref[...] + offs


def _tc_gemm(x4, nbr_pad, w4, b128):
    return pl.pallas_call(
        _tc_body,
        grid=(NB,),
        in_specs=[
            pl.BlockSpec((BLK, YW), lambda i: (i, 0)),
            pl.BlockSpec((KVOL, PACK * BLK), lambda i: (0, i)),
            pl.BlockSpec((KVOL, YW, YW), lambda i: (0, 0, 0)),
            pl.BlockSpec((1, YW), lambda i: (0, 0)),
        ],
        out_specs=[
            pl.BlockSpec((KVOL, BLK, YW), lambda i: (0, i, 0)),
            pl.BlockSpec((KVOL, PACK * BLK), lambda i: (0, i)),
        ],
        out_shape=[
            jax.ShapeDtypeStruct((KVOL, NPAD // PACK, YW), jnp.float32),
            jax.ShapeDtypeStruct((KVOL, NPAD), jnp.int32),
        ],
    )(x4, nbr_pad, w4, b128)


def _sc_gather_sum(y, idx):
    nc = 2
    mesh = plsc.VectorSubcoreMesh(
        core_axis_name="c", subcore_axis_name="s", num_cores=nc, num_subcores=16
    )

    y2d = y.reshape(KVOL * NPAD, COUT)
    idx1d = idx.reshape(KVOL * NPAD)

    @functools.partial(
        pl.kernel,
        out_type=jax.ShapeDtypeStruct((NPAD, COUT), jnp.float32),
        mesh=mesh,
        scratch_types=[
            pltpu.VMEM((KVOL * ROWS_PER_W,), jnp.int32),
            pltpu.VMEM((ROWS_PER_W, COUT), jnp.float32),
            pltpu.SemaphoreType.DMA,
            pltpu.SemaphoreType.DMA,
        ],
        compiler_params=pltpu.CompilerParams(use_tc_tiling_on_sc=False),
    )
    def sc_k(y_hbm, idx_hbm, out_hbm, idx_v, acc_v, sem, isem):
        wid = lax.axis_index("s") * nc + lax.axis_index("c")
        base = wid * ROWS_PER_W
        idescs = [
            pltpu.async_copy(
                idx_hbm.at[pl.ds(k * NPAD + base, ROWS_PER_W)],
                idx_v.at[pl.ds(k * ROWS_PER_W, ROWS_PER_W)],
                isem,
            )
            for k in range(KVOL)
        ]
        z = jnp.zeros((16,), jnp.float32)

        def zero_body(r, carry):
            acc_v[r, pl.ds(0, 16)] = z
            acc_v[r, pl.ds(16, 16)] = z
            return carry

        lax.fori_loop(0, ROWS_PER_W, zero_body, 0)
        for d in idescs:
            d.wait()
        descs = [
            pltpu.async_copy(
                y_hbm.at[idx_v.at[pl.ds(k * ROWS_PER_W, ROWS_PER_W)]],
                acc_v,
                sem,
                add=True,
            )
            for k in range(KVOL)
        ]
        for d in descs:
            d.wait()
        pltpu.sync_copy(acc_v, out_hbm.at[pl.ds(base, ROWS_PER_W)])

    return sc_k(y2d, idx1d)


def kernel(in_feature, nbr_idx, weight, bias):
    w = weight.reshape(COUT, CIN, KVOL).transpose(2, 1, 0)
    w4 = jnp.einsum("tu,kcd->ktcud", jnp.eye(PACK, dtype=w.dtype), w)
    w4 = w4.reshape(KVOL, YW, YW)
    b128 = jnp.tile(bias.reshape(1, COUT), (1, PACK))
    x4 = jnp.pad(in_feature, ((0, NPAD - N), (0, 0))).reshape(NPAD // PACK, YW)
    nbr_pad = jnp.pad(nbr_idx, ((0, 0), (0, NPAD - N)))
    y, idx = _tc_gemm(x4, nbr_pad, w4, b128)
    out = _sc_gather_sum(y, idx)
    return out[:N]


# speedup vs baseline: 4.0186x; 1.0054x over previous
import functools

import jax
import jax.numpy as jnp
from jax import lax
from jax.experimental import pallas as pl
from jax.experimental.pallas import tpu as pltpu
from jax.experimental.pallas import tpu_sc as plsc

N = 50000
CIN = 32
COUT = 32
KVOL = 27
PACK = 4
YW = PACK * COUT

NW = 32
ROWS_PER_W = 1664
NPAD = NW * ROWS_PER_W
BLK = 512
NB = NPAD // PACK // BLK


def _tc_body(x_ref, nbr_ref, w_ref, b_ref, y_ref, idx_ref):
    x = x_ref[...]
    for k in range(KVOL):
        y = jnp.dot(x, w_ref[k], preferred_element_type=jnp.float32)
        if k == 0:
            y = y + b_ref[...]
        y_ref[k] = y.astype(jnp.bfloat16)
    offs = lax.broadcasted_iota(jnp.int32, (KVOL, PACK * BLK), 0) * NPAD
    idx_ref[...] = nbr_ref[...] + offs


def _tc_gemm(x4, nbr_pad, w4, b128):
    return pl.pallas_call(
        _tc_body,
        grid=(NB,),
        in_specs=[
            pl.BlockSpec((BLK, YW), lambda i: (i, 0)),
            pl.BlockSpec((KVOL, PACK * BLK), lambda i: (0, i)),
            pl.BlockSpec((KVOL, YW, YW), lambda i: (0, 0, 0)),
            pl.BlockSpec((1, YW), lambda i: (0, 0)),
        ],
        out_specs=[
            pl.BlockSpec((KVOL, BLK, YW), lambda i: (0, i, 0)),
            pl.BlockSpec((KVOL, PACK * BLK), lambda i: (0, i)),
        ],
        out_shape=[
            jax.ShapeDtypeStruct((KVOL, NPAD // PACK, YW), jnp.bfloat16),
            jax.ShapeDtypeStruct((KVOL, NPAD), jnp.int32),
        ],
    )(x4, nbr_pad, w4, b128)


def _sc_gather_sum(y, idx):
    nc = 2
    mesh = plsc.VectorSubcoreMesh(
        core_axis_name="c", subcore_axis_name="s", num_cores=nc, num_subcores=16
    )

    y2d = y.reshape(KVOL * NPAD, COUT)
    idx1d = idx.reshape(KVOL * NPAD)

    @functools.partial(
        pl.kernel,
        out_type=jax.ShapeDtypeStruct((NPAD, COUT), jnp.bfloat16),
        mesh=mesh,
        scratch_types=[
            pltpu.VMEM((KVOL * ROWS_PER_W,), jnp.int32),
            pltpu.VMEM((ROWS_PER_W, COUT), jnp.bfloat16),
            pltpu.SemaphoreType.DMA,
            pltpu.SemaphoreType.DMA,
        ],
        compiler_params=pltpu.CompilerParams(use_tc_tiling_on_sc=False),
    )
    def sc_k(y_hbm, idx_hbm, out_hbm, idx_v, acc_v, sem, isem):
        wid = lax.axis_index("s") * nc + lax.axis_index("c")
        base = wid * ROWS_PER_W
        idescs = [
            pltpu.async_copy(
                idx_hbm.at[pl.ds(k * NPAD + base, ROWS_PER_W)],
                idx_v.at[pl.ds(k * ROWS_PER_W, ROWS_PER_W)],
                isem,
            )
            for k in range(KVOL)
        ]
        z = jnp.zeros((32,), jnp.bfloat16)

        def zero_body(r, carry):
            acc_v[r, :] = z
            return carry

        lax.fori_loop(0, ROWS_PER_W, zero_body, 0)
        for d in idescs:
            d.wait()
        descs = [
            pltpu.async_copy(
                y_hbm.at[idx_v.at[pl.ds(k * ROWS_PER_W, ROWS_PER_W)]],
                acc_v,
                sem,
                add=True,
            )
            for k in range(KVOL)
        ]
        for d in descs:
            d.wait()
        pltpu.sync_copy(acc_v, out_hbm.at[pl.ds(base, ROWS_PER_W)])

    return sc_k(y2d, idx1d)


def kernel(in_feature, nbr_idx, weight, bias):
    w = weight.reshape(COUT, CIN, KVOL).transpose(2, 1, 0)
    w4 = jnp.einsum("tu,kcd->ktcud", jnp.eye(PACK, dtype=w.dtype), w)
    w4 = w4.reshape(KVOL, YW, YW)
    b128 = jnp.tile(bias.reshape(1, COUT), (1, PACK))
    x4 = jnp.pad(in_feature, ((0, NPAD - N), (0, 0))).reshape(NPAD // PACK, YW)
    nbr_pad = jnp.pad(nbr_idx, ((0, 0), (0, NPAD - N)))
    y, idx = _tc_gemm(x4, nbr_pad, w4, b128)
    out = _sc_gather_sum(y, idx)
    return out[:N].astype(jnp.float32)
